# Initial kernel scaffold; baseline (speedup 1.0000x reference)
import jax
import jax.numpy as jnp
from jax.experimental import pallas as pl


def kernel(queries, train_data):
    raise NotImplementedError("write your pallas kernel here")


# speedup vs baseline: 1.9337x; 1.9337x over previous
import math

import jax
import jax.numpy as jnp
from jax.experimental import pallas as pl
from jax.experimental.pallas import tpu as pltpu

_H = 4.0
_INV_H2 = 1.0 / (_H * _H)


def _kde_tile(q_ref, t_ref, logb_ref, out_ref):
    qf = q_ref[...]
    q2 = jnp.sum(qf * qf, axis=1, keepdims=True)
    qs = (qf * _INV_H2).astype(jnp.bfloat16)
    s = jax.lax.dot_general(
        qs, t_ref[...],
        dimension_numbers=(((1,), (1,)), ((), ())),
        preferred_element_type=jnp.float32)
    e = jnp.exp(s + logb_ref[...])
    r = jnp.sum(e, axis=1, keepdims=True)
    d = q_ref.shape[1]
    nt = t_ref.shape[0]
    const = (-0.5 * d * math.log(2.0 * math.pi)
             - d * math.log(_H) - math.log(nt))
    out_ref[...] = jnp.log(r) - (0.5 * _INV_H2) * q2 + const


def kernel(queries, train_data):
    nq, d = queries.shape
    nt, _ = train_data.shape
    qb = 256
    t_bf = train_data.astype(jnp.bfloat16)
    t2 = jnp.sum(train_data * train_data, axis=1)
    logb = ((-0.5 * _INV_H2) * t2)[None, :]
    out = pl.pallas_call(
        _kde_tile,
        grid=(nq // qb,),
        in_specs=[
            pl.BlockSpec((qb, d), lambda i: (i, 0)),
            pl.BlockSpec((nt, d), lambda i: (0, 0)),
            pl.BlockSpec((1, nt), lambda i: (0, 0)),
        ],
        out_specs=pl.BlockSpec((qb, 1), lambda i: (i, 0)),
        out_shape=jax.ShapeDtypeStruct((nq, 1), jnp.float32),
    )(queries, t_bf, logb)
    return out[:, 0]
